# baseline (speedup 1.0000x reference)
import jax
import jax.numpy as jnp
from jax.experimental import pallas as pl

_B = 1024

_TOPK_SCALE = 2.5


def _router_block(x_ref, w_ref, b_ref, idx_ref, wgt_ref):
    x = x_ref[...]
    w = w_ref[...]
    bias = b_ref[...]

    h = jax.lax.dot_general(w, x, (((1,), (1,)), ((), ())),
                            preferred_element_type=jnp.float32)
    s = jax.nn.sigmoid(h) + bias

    B = s.shape[1]
    iota = jax.lax.broadcasted_iota(jnp.int32, (8, B), 0)
    gidx = iota >> 1

    p = jnp.concatenate(
        [s[1:2], s[0:1], s[3:4], s[2:3], s[5:6], s[4:5], s[7:8], s[6:7]],
        axis=0)
    gw = s + p

    NEG = jnp.float32(-1.0)

    g1v = jnp.max(gw, axis=0, keepdims=True)
    g1 = jnp.min(jnp.where(gw == g1v, gidx, 4), axis=0, keepdims=True)
    gw2 = jnp.where(gidx == g1, NEG, gw)
    g2v = jnp.max(gw2, axis=0, keepdims=True)
    g2 = jnp.min(jnp.where(gw2 == g2v, gidx, 4), axis=0, keepdims=True)
    sel = (gidx == g1) | (gidx == g2)

    m = jnp.where(sel, s, 0.0)
    v1 = jnp.max(m, axis=0, keepdims=True)
    e1 = jnp.min(jnp.where(m == v1, iota, 8), axis=0, keepdims=True)
    m2 = jnp.where(iota == e1, NEG, m)
    v2 = jnp.max(m2, axis=0, keepdims=True)
    e2 = jnp.min(jnp.where(m2 == v2, iota, 8), axis=0, keepdims=True)

    scale = _TOPK_SCALE / (v1 + v2)
    zi = jnp.zeros((6, B), jnp.int32)
    zf = jnp.zeros((6, B), jnp.float32)
    idx8 = jnp.concatenate([e1, e2, zi], axis=0).T
    wgt8 = jnp.concatenate([v1 * scale, v2 * scale, zf], axis=0).T
    idx_ref[...] = idx8[:, :2]
    wgt_ref[...] = wgt8[:, :2]


def kernel(hidden_tensor, weight, scores_bias):
    T, D = hidden_tensor.shape
    E = weight.shape[0]
    idx_t, wgt_t = pl.pallas_call(
        _router_block,
        grid=(T // _B,),
        in_specs=[
            pl.BlockSpec((_B, D), lambda i: (i, 0)),
            pl.BlockSpec((E, D), lambda i: (0, 0)),
            pl.BlockSpec((E, 1), lambda i: (0, 0)),
        ],
        out_specs=[
            pl.BlockSpec((_B, 2), lambda i: (i, 0)),
            pl.BlockSpec((_B, 2), lambda i: (i, 0)),
        ],
        out_shape=[
            jax.ShapeDtypeStruct((T, 2), jnp.int32),
            jax.ShapeDtypeStruct((T, 2), jnp.float32),
        ],
    )(hidden_tensor, weight, scores_bias.reshape(E, 1))
    return (idx_t, wgt_t)


# speedup vs baseline: 1.0621x; 1.0621x over previous
import functools

import jax
import jax.numpy as jnp
from jax import lax
from jax.experimental import pallas as pl
from jax.experimental.pallas import tpu as pltpu
from jax.experimental.pallas import tpu_sc as plsc

_B = 1024
_NW = 32
_L = 16

_TOPK_SCALE = 2.5


def _scores_block(x_ref, w_ref, b_ref, s_ref):
    x = x_ref[...]
    w = w_ref[...]
    bias = b_ref[...]
    h = jax.lax.dot_general(w, x, (((1,), (1,)), ((), ())),
                            preferred_element_type=jnp.float32)
    s_ref[...] = jax.nn.sigmoid(h) + bias


def _tc_scores(hidden_tensor, weight, scores_bias):
    T, D = hidden_tensor.shape
    E = weight.shape[0]
    return pl.pallas_call(
        _scores_block,
        grid=(T // _B,),
        in_specs=[
            pl.BlockSpec((_B, D), lambda i: (i, 0)),
            pl.BlockSpec((E, D), lambda i: (0, 0)),
            pl.BlockSpec((E, 1), lambda i: (0, 0)),
        ],
        out_specs=pl.BlockSpec((E, _B), lambda i: (0, i)),
        out_shape=jax.ShapeDtypeStruct((E, T), jnp.float32),
    )(hidden_tensor, weight, scores_bias.reshape(E, 1))


def _first_idx(vals, target, n):
    out = jnp.full((_L,), n - 1, jnp.int32)
    for k in range(n - 2, -1, -1):
        out = jnp.where(vals[k] == target, jnp.int32(k), out)
    return out


def _route16(s):
    neg = jnp.float32(-1.0)
    gw = [s[0] + s[1], s[2] + s[3], s[4] + s[5], s[6] + s[7]]
    gmax = jnp.maximum(jnp.maximum(gw[0], gw[1]), jnp.maximum(gw[2], gw[3]))
    g1 = _first_idx(gw, gmax, 4)
    gwb = [jnp.where(g1 == k, neg, gw[k]) for k in range(4)]
    gmax2 = jnp.maximum(jnp.maximum(gwb[0], gwb[1]),
                        jnp.maximum(gwb[2], gwb[3]))
    g2 = _first_idx(gwb, gmax2, 4)

    m = [jnp.where((g1 == (e >> 1)) | (g2 == (e >> 1)), s[e], 0.0)
         for e in range(8)]
    v1 = m[0]
    for e in range(1, 8):
        v1 = jnp.maximum(v1, m[e])
    e1 = _first_idx(m, v1, 8)
    mb = [jnp.where(e1 == e, neg, m[e]) for e in range(8)]
    v2 = mb[0]
    for e in range(1, 8):
        v2 = jnp.maximum(v2, mb[e])
    e2 = _first_idx(mb, v2, 8)

    scale = _TOPK_SCALE / (v1 + v2)
    return e1, e2, v1 * scale, v2 * scale


def _sc_route_kernel(T):
    C = T // _NW
    mesh = plsc.VectorSubcoreMesh(core_axis_name="c", subcore_axis_name="s")

    @functools.partial(
        pl.kernel, mesh=mesh,
        out_type=[
            jax.ShapeDtypeStruct((T,), jnp.int32),
            jax.ShapeDtypeStruct((T,), jnp.int32),
            jax.ShapeDtypeStruct((T,), jnp.float32),
            jax.ShapeDtypeStruct((T,), jnp.float32),
        ],
        scratch_types=[
            pltpu.VMEM((8, C), jnp.float32),
            pltpu.VMEM((C,), jnp.int32),
            pltpu.VMEM((C,), jnp.int32),
            pltpu.VMEM((C,), jnp.float32),
            pltpu.VMEM((C,), jnp.float32),
        ],
    )
    def route(s_hbm, e1_hbm, e2_hbm, w1_hbm, w2_hbm,
              s_v, e1_v, e2_v, w1_v, w2_v):
        wid = lax.axis_index("s") * 2 + lax.axis_index("c")
        base = wid * C
        pltpu.sync_copy(s_hbm.at[:, pl.ds(base, C)], s_v)

        def body(i, carry):
            off = i * _L
            s = [s_v[e, pl.ds(off, _L)] for e in range(8)]
            e1, e2, w1, w2 = _route16(s)
            e1_v[pl.ds(off, _L)] = e1
            e2_v[pl.ds(off, _L)] = e2
            w1_v[pl.ds(off, _L)] = w1
            w2_v[pl.ds(off, _L)] = w2
            return carry

        lax.fori_loop(0, C // _L, body, 0)
        pltpu.sync_copy(e1_v, e1_hbm.at[pl.ds(base, C)])
        pltpu.sync_copy(e2_v, e2_hbm.at[pl.ds(base, C)])
        pltpu.sync_copy(w1_v, w1_hbm.at[pl.ds(base, C)])
        pltpu.sync_copy(w2_v, w2_hbm.at[pl.ds(base, C)])

    return route


def kernel(hidden_tensor, weight, scores_bias):
    T, _ = hidden_tensor.shape
    scores = _tc_scores(hidden_tensor, weight, scores_bias)
    e1, e2, w1, w2 = _sc_route_kernel(T)(scores)
    return (jnp.stack([e1, e2], axis=1), jnp.stack([w1, w2], axis=1))


# speedup vs baseline: 1.3034x; 1.2272x over previous
import jax
import jax.numpy as jnp
from jax.experimental import pallas as pl

_B = 1024

_TOPK_SCALE = 2.5


def _router_block(x_ref, w_ref, b_ref, out_ref):
    x = x_ref[...]
    w = w_ref[...]
    bias = b_ref[...]

    h = jax.lax.dot_general(w, x, (((1,), (1,)), ((), ())),
                            preferred_element_type=jnp.float32)
    s = jax.nn.sigmoid(h) + bias

    B = s.shape[1]
    iota = jax.lax.broadcasted_iota(jnp.int32, (8, B), 0)
    gidx = iota >> 1

    p = jnp.concatenate(
        [s[1:2], s[0:1], s[3:4], s[2:3], s[5:6], s[4:5], s[7:8], s[6:7]],
        axis=0)
    gw = s + p

    NEG = jnp.float32(-1.0)

    g1v = jnp.max(gw, axis=0, keepdims=True)
    g1 = jnp.min(jnp.where(gw == g1v, gidx, 4), axis=0, keepdims=True)
    gw2 = jnp.where(gidx == g1, NEG, gw)
    g2v = jnp.max(gw2, axis=0, keepdims=True)
    g2 = jnp.min(jnp.where(gw2 == g2v, gidx, 4), axis=0, keepdims=True)
    sel = (gidx == g1) | (gidx == g2)

    m = jnp.where(sel, s, 0.0)
    v1 = jnp.max(m, axis=0, keepdims=True)
    e1 = jnp.min(jnp.where(m == v1, iota, 8), axis=0, keepdims=True)
    m2 = jnp.where(iota == e1, NEG, m)
    v2 = jnp.max(m2, axis=0, keepdims=True)
    e2 = jnp.min(jnp.where(m2 == v2, iota, 8), axis=0, keepdims=True)

    scale = _TOPK_SCALE / (v1 + v2)
    w1b = jax.lax.bitcast_convert_type(v1 * scale, jnp.int32)
    w2b = jax.lax.bitcast_convert_type(v2 * scale, jnp.int32)
    out_ref[...] = jnp.concatenate([e1, e2, w1b, w2b], axis=0)


def kernel(hidden_tensor, weight, scores_bias):
    T, D = hidden_tensor.shape
    E = weight.shape[0]
    packed = pl.pallas_call(
        _router_block,
        grid=(T // _B,),
        in_specs=[
            pl.BlockSpec((_B, D), lambda i: (i, 0)),
            pl.BlockSpec((E, D), lambda i: (0, 0)),
            pl.BlockSpec((E, 1), lambda i: (0, 0)),
        ],
        out_specs=pl.BlockSpec((4, _B), lambda i: (0, i)),
        out_shape=jax.ShapeDtypeStruct((4, T), jnp.int32),
    )(hidden_tensor, weight, scores_bias.reshape(E, 1))
    o = packed.T
    return (o[:, :2], jax.lax.bitcast_convert_type(o[:, 2:], jnp.float32))


# speedup vs baseline: 1.3588x; 1.0425x over previous
import jax
import jax.numpy as jnp
from jax.experimental import pallas as pl

_B = 1024

_TOPK_SCALE = 2.5


def _router_block(x1_ref, x2_ref, w_ref, b_ref, idx_ref, wgt_ref):
    x1 = x1_ref[...]
    x2 = x2_ref[...]
    w = w_ref[...]
    bias = b_ref[...]
    Dh = x1.shape[1]

    h = (jax.lax.dot_general(w[:, :Dh], x1, (((1,), (1,)), ((), ())),
                             preferred_element_type=jnp.float32)
         + jax.lax.dot_general(w[:, Dh:], x2, (((1,), (1,)), ((), ())),
                               preferred_element_type=jnp.float32))
    s = jax.nn.sigmoid(h) + bias

    B = s.shape[1]
    iota = jax.lax.broadcasted_iota(jnp.int32, (8, B), 0)
    gidx = iota >> 1

    p = jnp.concatenate(
        [s[1:2], s[0:1], s[3:4], s[2:3], s[5:6], s[4:5], s[7:8], s[6:7]],
        axis=0)
    gw = s + p

    NEG = jnp.float32(-1.0)

    g1v = jnp.max(gw, axis=0, keepdims=True)
    g1 = jnp.min(jnp.where(gw == g1v, gidx, 4), axis=0, keepdims=True)
    gw2 = jnp.where(gidx == g1, NEG, gw)
    g2v = jnp.max(gw2, axis=0, keepdims=True)
    g2 = jnp.min(jnp.where(gw2 == g2v, gidx, 4), axis=0, keepdims=True)
    sel = (gidx == g1) | (gidx == g2)

    m = jnp.where(sel, s, 0.0)
    v1 = jnp.max(m, axis=0, keepdims=True)
    e1 = jnp.min(jnp.where(m == v1, iota, 8), axis=0, keepdims=True)
    m2 = jnp.where(iota == e1, NEG, m)
    v2 = jnp.max(m2, axis=0, keepdims=True)
    e2 = jnp.min(jnp.where(m2 == v2, iota, 8), axis=0, keepdims=True)

    scale = _TOPK_SCALE / (v1 + v2)
    idx_ref[...] = jnp.concatenate([e1, e2], axis=0)
    wgt_ref[...] = jnp.concatenate([v1 * scale, v2 * scale], axis=0)


def kernel(hidden_tensor, weight, scores_bias):
    T, D = hidden_tensor.shape
    E = weight.shape[0]
    idx_t, wgt_t = pl.pallas_call(
        _router_block,
        grid=(T // _B,),
        in_specs=[
            pl.BlockSpec((_B, D // 2), lambda i: (i, 0)),
            pl.BlockSpec((_B, D // 2), lambda i: (i, 1)),
            pl.BlockSpec((E, D), lambda i: (0, 0)),
            pl.BlockSpec((E, 1), lambda i: (0, 0)),
        ],
        out_specs=[
            pl.BlockSpec((2, _B), lambda i: (0, i)),
            pl.BlockSpec((2, _B), lambda i: (0, i)),
        ],
        out_shape=[
            jax.ShapeDtypeStruct((2, T), jnp.int32),
            jax.ShapeDtypeStruct((2, T), jnp.float32),
        ],
    )(hidden_tensor, hidden_tensor, weight, scores_bias.reshape(E, 1))
    return (idx_t.T, wgt_t.T)


# speedup vs baseline: 1.3623x; 1.0026x over previous
import jax
import jax.numpy as jnp
from jax.experimental import pallas as pl

_B = 1024

_TOPK_SCALE = 2.5


def _router_block(x_ref, w_ref, b_ref, idx_ref, wgt_ref):
    x = x_ref[...]
    w = w_ref[...]
    bias = b_ref[...]

    h = jax.lax.dot_general(w, x, (((1,), (1,)), ((), ())),
                            preferred_element_type=jnp.float32)
    s = jax.nn.sigmoid(h) + bias

    B = s.shape[1]
    iota = jax.lax.broadcasted_iota(jnp.int32, (8, B), 0)
    gidx = iota >> 1

    p = jnp.concatenate(
        [s[1:2], s[0:1], s[3:4], s[2:3], s[5:6], s[4:5], s[7:8], s[6:7]],
        axis=0)
    gw = s + p

    NEG = jnp.float32(-1.0)

    g1v = jnp.max(gw, axis=0, keepdims=True)
    g1 = jnp.min(jnp.where(gw == g1v, gidx, 4), axis=0, keepdims=True)
    gw2 = jnp.where(gidx == g1, NEG, gw)
    g2v = jnp.max(gw2, axis=0, keepdims=True)
    g2 = jnp.min(jnp.where(gw2 == g2v, gidx, 4), axis=0, keepdims=True)
    sel = (gidx == g1) | (gidx == g2)

    m = jnp.where(sel, s, 0.0)
    v1 = jnp.max(m, axis=0, keepdims=True)
    e1 = jnp.min(jnp.where(m == v1, iota, 8), axis=0, keepdims=True)
    m2 = jnp.where(iota == e1, NEG, m)
    v2 = jnp.max(m2, axis=0, keepdims=True)
    e2 = jnp.min(jnp.where(m2 == v2, iota, 8), axis=0, keepdims=True)

    scale = _TOPK_SCALE / (v1 + v2)
    idx_ref[...] = jnp.concatenate([e1, e2], axis=0)
    wgt_ref[...] = jnp.concatenate([v1 * scale, v2 * scale], axis=0)


def kernel(hidden_tensor, weight, scores_bias):
    T, D = hidden_tensor.shape
    E = weight.shape[0]
    idx_t, wgt_t = pl.pallas_call(
        _router_block,
        grid=(T // _B,),
        in_specs=[
            pl.BlockSpec((_B, D), lambda i: (i, 0)),
            pl.BlockSpec((E, D), lambda i: (0, 0)),
            pl.BlockSpec((E, 1), lambda i: (0, 0)),
        ],
        out_specs=[
            pl.BlockSpec((2, _B), lambda i: (0, i)),
            pl.BlockSpec((2, _B), lambda i: (0, i)),
        ],
        out_shape=[
            jax.ShapeDtypeStruct((2, T), jnp.int32),
            jax.ShapeDtypeStruct((2, T), jnp.float32),
        ],
    )(hidden_tensor, weight, scores_bias.reshape(E, 1))
    return (idx_t.T, wgt_t.T)
